# baseline (speedup 1.0000x reference)
import functools

import jax
import jax.numpy as jnp
from jax import lax
from jax.experimental import pallas as pl
from jax.experimental.pallas import tpu as pltpu
from jax.experimental.pallas import tpu_sc as plsc

_N = 10000
_E = 320000
_D = 128
_H = 64
_C = 40
_K = 10

_NP = 10240
_NA = _NP + 16
_CP = 48
_NT = 16
_NCH = _NP // _NT
_ECH = _E // _NT
_EB = 128
_NEB = 158
_DB = 128
_NDB = _NCH // _DB


def _rsqrt(x):
  i = lax.bitcast_convert_type(x, jnp.int32)
  i = jnp.int32(0x5F3759DF) - (i >> 1)
  y = lax.bitcast_convert_type(i, jnp.float32)
  h = x * 0.5
  for _ in range(3):
    y = y * (1.5 - h * y * y)
  return y


def _sc_body(h_hbm, row_hbm, col_hbm, temp_hbm, out_hbm, u_hbm,
             acc_s, deg_s,
             rv0, rv1, cv0, cv1, gbuf0, gbuf1, abuf, hidc,
             dis_v, dtmp_v, ones_v, temp_v,
             is0, is1, gs0, gs1, ss0, ss1):
  wid = lax.axis_index("s")
  nbase = wid * _NCH

  pltpu.sync_copy(temp_hbm, temp_v)

  one16 = jnp.full((16,), 1.0, jnp.float32)

  def fill_ones(i, c):
    ones_v[pl.ds(i * 16, 16)] = one16
    return c
  lax.fori_loop(0, _EB // 16, fill_ones, 0)

  def fill_deg(i, c):
    dtmp_v[pl.ds(i * 16, 16)] = one16
    return c
  lax.fori_loop(0, _NCH // 16, fill_deg, 0)
  pltpu.sync_copy(dtmp_v, deg_s.at[pl.ds(nbase, _NCH)])
  plsc.subcore_barrier()

  def deg_batch(b, c):
    pltpu.sync_copy(col_hbm.at[wid].at[b], cv0)
    pltpu.sync_copy(ones_v, deg_s.at[cv0], add=True)
    return c
  lax.fori_loop(0, _NEB, deg_batch, 0)
  plsc.subcore_barrier()

  pltpu.sync_copy(deg_s.at[pl.ds(nbase, _NCH)], dtmp_v)

  def mk_dis(i, c):
    d = dtmp_v[pl.ds(i * 16, 16)]
    dis_v[pl.ds(i * 16, 16)] = _rsqrt(d)
    return c
  lax.fori_loop(0, _NCH // 16, mk_dis, 0)

  t0 = plsc.load_gather(temp_v, [jnp.full((16,), 1, jnp.int32)])
  for blk in range(_NDB):
    base = nbase + blk * _DB
    pltpu.sync_copy(h_hbm.at[pl.ds(base, _DB)], abuf)

    def init_node(jj, c, blk=blk):
      idxv = jnp.full((16,), jj + blk * _DB, jnp.int32)
      dsp = plsc.load_gather(dis_v, [idxv])
      for cc in range(_CP // 16):
        hcol = abuf[jj, pl.ds(16 * cc, 16)]
        hidc[jj + blk * _DB, pl.ds(16 * cc, 16)] = t0 * hcol
        abuf[jj, pl.ds(16 * cc, 16)] = dsp * hcol
      return c
    lax.fori_loop(0, _DB, init_node, 0)
    pltpu.sync_copy(abuf, u_hbm.at[pl.ds(base, _DB)])
    pltpu.sync_copy(abuf, acc_s.at[pl.ds(base, _DB)])
  plsc.subcore_barrier()

  for k in range(_K):
    pltpu.sync_copy(row_hbm.at[wid].at[0], rv0)
    pltpu.sync_copy(col_hbm.at[wid].at[0], cv0)
    pltpu.async_copy(row_hbm.at[wid].at[1], rv1, is1)
    pltpu.async_copy(col_hbm.at[wid].at[1], cv1, is1)
    pltpu.async_copy(u_hbm.at[rv0], gbuf0, gs0)

    def edge_pair(i, c):
      b1 = 2 * i + 1
      b2 = 2 * i + 2
      b3 = 2 * i + 3
      pltpu.make_async_copy(u_hbm.at[rv0], gbuf0, gs0).wait()
      pltpu.async_copy(gbuf0, acc_s.at[cv0], ss0, add=True)
      pltpu.make_async_copy(row_hbm.at[wid].at[b1], rv1, is1).wait()
      pltpu.make_async_copy(col_hbm.at[wid].at[b1], cv1, is1).wait()
      pltpu.async_copy(u_hbm.at[rv1], gbuf1, gs1)
      pltpu.make_async_copy(gbuf0, acc_s.at[cv0], ss0).wait()

      @pl.when(b2 < _NEB)
      def _():
        pltpu.async_copy(row_hbm.at[wid].at[b2], rv0, is0)
        pltpu.async_copy(col_hbm.at[wid].at[b2], cv0, is0)
      pltpu.make_async_copy(u_hbm.at[rv1], gbuf1, gs1).wait()
      pltpu.async_copy(gbuf1, acc_s.at[cv1], ss1, add=True)

      @pl.when(b2 < _NEB)
      def _():
        pltpu.make_async_copy(row_hbm.at[wid].at[b2], rv0, is0).wait()
        pltpu.make_async_copy(col_hbm.at[wid].at[b2], cv0, is0).wait()
        pltpu.async_copy(u_hbm.at[rv0], gbuf0, gs0)
      pltpu.make_async_copy(gbuf1, acc_s.at[cv1], ss1).wait()

      @pl.when(b3 < _NEB)
      def _():
        pltpu.async_copy(row_hbm.at[wid].at[b3], rv1, is1)
        pltpu.async_copy(col_hbm.at[wid].at[b3], cv1, is1)
      return c
    lax.fori_loop(0, _NEB // 2, edge_pair, 0)
    plsc.subcore_barrier()

    tk = plsc.load_gather(temp_v, [jnp.full((16,), k + 2, jnp.int32)])
    for blk in range(_NDB):
      base = nbase + blk * _DB
      pltpu.sync_copy(acc_s.at[pl.ds(base, _DB)], abuf)

      def dense_node(jj, c2, blk=blk, tk=tk):
        idxv = jnp.full((16,), jj + blk * _DB, jnp.int32)
        dsp = plsc.load_gather(dis_v, [idxv])
        d2sp = dsp * dsp
        tdsp = dsp * tk
        for cc in range(_CP // 16):
          w = abuf[jj, pl.ds(16 * cc, 16)]
          nn = jj + blk * _DB
          hidc[nn, pl.ds(16 * cc, 16)] = (
              hidc[nn, pl.ds(16 * cc, 16)] + tdsp * w)
          abuf[jj, pl.ds(16 * cc, 16)] = d2sp * w
        return c2
      lax.fori_loop(0, _DB, dense_node, 0)
      if k < _K - 1:
        pltpu.sync_copy(abuf, u_hbm.at[pl.ds(base, _DB)])
        pltpu.sync_copy(abuf, acc_s.at[pl.ds(base, _DB)])
    if k < _K - 1:
      plsc.subcore_barrier()

  pltpu.sync_copy(hidc, out_hbm.at[pl.ds(nbase, _NCH)])


def _sc_propagate(h_pad, row3, col3, temp16, interpret=False):
  mesh = plsc.VectorSubcoreMesh(
      core_axis_name="c", subcore_axis_name="s", num_cores=1,
      num_subcores=_NT)
  f = pl.kernel(
      _sc_body,
      out_type=(jax.ShapeDtypeStruct((_NP, _CP), jnp.float32),
                jax.ShapeDtypeStruct((_NP, _CP), jnp.float32)),
      mesh=mesh,
      scratch_types=[
          pltpu.VMEM_SHARED((_NA, _CP), jnp.float32),
          pltpu.VMEM_SHARED((_NA,), jnp.float32),
          pltpu.VMEM((_EB,), jnp.int32),
          pltpu.VMEM((_EB,), jnp.int32),
          pltpu.VMEM((_EB,), jnp.int32),
          pltpu.VMEM((_EB,), jnp.int32),
          pltpu.VMEM((_EB, _CP), jnp.float32),
          pltpu.VMEM((_EB, _CP), jnp.float32),
          pltpu.VMEM((_DB, _CP), jnp.float32),
          pltpu.VMEM((_NCH, _CP), jnp.float32),
          pltpu.VMEM((_NCH,), jnp.float32),
          pltpu.VMEM((_NCH,), jnp.float32),
          pltpu.VMEM((_EB,), jnp.float32),
          pltpu.VMEM((16,), jnp.float32),
          pltpu.SemaphoreType.DMA,
          pltpu.SemaphoreType.DMA,
          pltpu.SemaphoreType.DMA,
          pltpu.SemaphoreType.DMA,
          pltpu.SemaphoreType.DMA,
          pltpu.SemaphoreType.DMA,
      ],
      compiler_params=pltpu.CompilerParams(needs_layout_passes=False,
                                           use_tc_tiling_on_sc=False),
      interpret=interpret,
  )
  return f(h_pad, row3, col3, temp16)[0]


def _tc_dense(x_pad, W1, b1, W2p, b2p):
  def body(x_ref, w1_ref, b1_ref, w2_ref, b2_ref, o_ref):
    h1 = jnp.dot(x_ref[...], w1_ref[...],
                 preferred_element_type=jnp.float32,
                 precision=lax.Precision.HIGHEST) + b1_ref[...]
    h1 = jnp.maximum(h1, 0.0)
    o_ref[...] = jnp.dot(h1, w2_ref[...],
                         preferred_element_type=jnp.float32,
                         precision=lax.Precision.HIGHEST) + b2_ref[...]

  return pl.pallas_call(
      body,
      grid=(_NP // 256,),
      in_specs=[
          pl.BlockSpec((256, _D), lambda i: (i, 0)),
          pl.BlockSpec((_D, _H), lambda i: (0, 0)),
          pl.BlockSpec((1, _H), lambda i: (0, 0)),
          pl.BlockSpec((_H, _CP), lambda i: (0, 0)),
          pl.BlockSpec((1, _CP), lambda i: (0, 0)),
      ],
      out_specs=pl.BlockSpec((256, _CP), lambda i: (i, 0)),
      out_shape=jax.ShapeDtypeStruct((_NP, _CP), jnp.float32),
  )(x_pad, W1, b1, W2p, b2p)


def _tc_logsoftmax(hid):
  def body(h_ref, o_ref):
    v = h_ref[:, :_C]
    m = jnp.max(v, axis=1, keepdims=True)
    e = jnp.exp(v - m)
    s = jnp.sum(e, axis=1, keepdims=True)
    o_ref[...] = v - m - jnp.log(s)

  return pl.pallas_call(
      body,
      grid=(10,),
      in_specs=[pl.BlockSpec((1000, _CP), lambda i: (i, 0))],
      out_specs=pl.BlockSpec((1000, _C), lambda i: (i, 0)),
      out_shape=jax.ShapeDtypeStruct((_N, _C), jnp.float32),
  )(hid)


@jax.jit
def kernel(x, edge_index, W1, b1, W2, b2, temp):
  x_pad = jnp.zeros((_NP, _D), jnp.float32).at[:_N].set(x)
  W2p = jnp.zeros((_H, _CP), jnp.float32).at[:, :_C].set(W2)
  b2p = jnp.zeros((1, _CP), jnp.float32).at[0, :_C].set(b2)
  row_t = edge_index[0].astype(jnp.int32).reshape(_NT, _ECH)
  col_t = edge_index[1].astype(jnp.int32).reshape(_NT, _ECH)
  pad = _NEB * _EB - _ECH
  row3 = jnp.pad(row_t, ((0, 0), (0, pad)),
                 constant_values=0).reshape(_NT, _NEB, _EB)
  col3 = jnp.pad(col_t, ((0, 0), (0, pad)),
                 constant_values=_NP).reshape(_NT, _NEB, _EB)
  temp16 = jnp.zeros((16,), jnp.float32).at[1:_K + 2].set(temp)

  h = _tc_dense(x_pad, W1, b1.reshape(1, _H), W2p, b2p)
  hid = _sc_propagate(h, row3, col3, temp16)
  return _tc_logsoftmax(hid)


# speedup vs baseline: 1.9869x; 1.9869x over previous
import functools

import jax
import jax.numpy as jnp
from jax import lax
from jax.experimental import pallas as pl
from jax.experimental.pallas import tpu as pltpu
from jax.experimental.pallas import tpu_sc as plsc

_N = 10000
_E = 320000
_D = 128
_H = 64
_C = 40
_K = 10

_NP = 10240
_NA = _NP + 16
_CP = 48
_NT = 16
_NCH = _NP // _NT
_ECH = _E // _NT
_EB = 128
_NSB = 10
_NB = _NSB * 16
_DB = 128
_NDB = _NCH // _DB


def _rsqrt(x):
  i = lax.bitcast_convert_type(x, jnp.int32)
  i = jnp.int32(0x5F3759DF) - (i >> 1)
  y = lax.bitcast_convert_type(i, jnp.float32)
  h = x * 0.5
  for _ in range(3):
    y = y * (1.5 - h * y * y)
  return y


def _sc_body(h_hbm, pk_hbm, temp_hbm, out_hbm,
             u_s, acc_s, deg_s,
             ib0, ib1, g0, g1, g2, g3, hidc,
             dis_v, dtmp_v, ones_v, temp_v,
             ise0, ise1, gs0, gs1, gs2, gs3, ss0, ss1, ss2, ss3):
  wid = lax.axis_index("s")
  nbase = wid * _NCH
  gb = (g0, g1, g2, g3)
  gsm = (gs0, gs1, gs2, gs3)
  ssm = (ss0, ss1, ss2, ss3)
  ibs = (ib0, ib1)
  ises = (ise0, ise1)

  pltpu.sync_copy(temp_hbm, temp_v)

  one16 = jnp.full((16,), 1.0, jnp.float32)

  def fill_ones(i, c):
    ones_v[pl.ds(i * 16, 16)] = one16
    return c
  lax.fori_loop(0, _EB // 16, fill_ones, 0)

  def fill_deg(i, c):
    dtmp_v[pl.ds(i * 16, 16)] = one16
    return c
  lax.fori_loop(0, _NCH // 16, fill_deg, 0)
  pltpu.sync_copy(dtmp_v, deg_s.at[pl.ds(nbase, _NCH)])
  plsc.subcore_barrier()

  def deg_sb(sb, c):
    pltpu.sync_copy(pk_hbm.at[wid].at[sb], ib0)
    for j in range(16):
      pltpu.async_copy(ones_v, deg_s.at[ib0.at[16 + j]], ss0, add=True)
    for j in range(16):
      pltpu.make_async_copy(ones_v, deg_s.at[ib0.at[16 + j]], ss0).wait()
    return c
  lax.fori_loop(0, _NSB, deg_sb, 0)
  plsc.subcore_barrier()

  pltpu.sync_copy(deg_s.at[pl.ds(nbase, _NCH)], dtmp_v)

  def mk_dis(i, c):
    d = dtmp_v[pl.ds(i * 16, 16)]
    dis_v[pl.ds(i * 16, 16)] = _rsqrt(d)
    return c
  lax.fori_loop(0, _NCH // 16, mk_dis, 0)

  t0 = plsc.load_gather(temp_v, [jnp.full((16,), 1, jnp.int32)])
  for blk in range(_NDB):
    base = nbase + blk * _DB
    pltpu.sync_copy(h_hbm.at[pl.ds(base, _DB)], g0)

    def init_node(jj, c, blk=blk):
      idxv = jnp.full((16,), jj + blk * _DB, jnp.int32)
      dsp = plsc.load_gather(dis_v, [idxv])
      for cc in range(_CP // 16):
        hcol = g0[jj, pl.ds(16 * cc, 16)]
        hidc[jj + blk * _DB, pl.ds(16 * cc, 16)] = t0 * hcol
        g0[jj, pl.ds(16 * cc, 16)] = dsp * hcol
      return c
    lax.fori_loop(0, _DB, init_node, 0)
    pltpu.sync_copy(g0, u_s.at[pl.ds(base, _DB)])
    pltpu.sync_copy(g0, acc_s.at[pl.ds(base, _DB)])
  plsc.subcore_barrier()

  def hop(k, c):
    pltpu.sync_copy(pk_hbm.at[wid].at[0], ib0)
    pltpu.async_copy(u_s.at[ib0.at[0]], g0, gs0)
    pltpu.async_copy(u_s.at[ib0.at[1]], g1, gs1)

    def sb_pair(p, c2):
      for q in range(2):
        sb = 2 * p + q
        ibA = ibs[q]
        ibB = ibs[1 - q]
        iseB = ises[1 - q]
        for j in range(16):
          m = j % 4
          pltpu.make_async_copy(u_s.at[ibA.at[j]], gb[m], gsm[m]).wait()
          pltpu.async_copy(gb[m], acc_s.at[ibA.at[16 + j]], ssm[m],
                           add=True)
          if j == 2:
            @pl.when(sb < _NSB - 1)
            def _():
              pltpu.async_copy(pk_hbm.at[wid].at[sb + 1], ibB, iseB)
          nj = j + 2
          m2 = nj % 4
          gg = sb * 16 + j

          @pl.when(gg + 2 < _NB)
          def _(nj=nj, m2=m2, gg=gg, ibA=ibA, ibB=ibB, iseB=iseB):
            @pl.when(gg >= 2)
            def __():
              pltpu.make_async_copy(gb[m2], acc_s.at[ibA.at[16]],
                                    ssm[m2]).wait()
            if nj < 16:
              pltpu.async_copy(u_s.at[ibA.at[nj]], gb[m2], gsm[m2])
            else:
              if nj == 16:
                pltpu.make_async_copy(pk_hbm.at[wid].at[0], ibB,
                                      iseB).wait()
              pltpu.async_copy(u_s.at[ibB.at[nj - 16]], gb[m2], gsm[m2])
      return c2
    lax.fori_loop(0, _NSB // 2, sb_pair, 0)
    pltpu.make_async_copy(g0, acc_s.at[ib0.at[16]], ss0).wait()
    pltpu.make_async_copy(g1, acc_s.at[ib0.at[16]], ss1).wait()
    pltpu.make_async_copy(g2, acc_s.at[ib0.at[16]], ss2).wait()
    pltpu.make_async_copy(g3, acc_s.at[ib0.at[16]], ss3).wait()
    plsc.subcore_barrier()

    tk = plsc.load_gather(temp_v, [jnp.full((16,), 2, jnp.int32) + k])
    for blk in range(_NDB):
      base = nbase + blk * _DB
      pltpu.sync_copy(acc_s.at[pl.ds(base, _DB)], g0)

      def dense_node(jj, c2, blk=blk, tk=tk):
        idxv = jnp.full((16,), jj + blk * _DB, jnp.int32)
        dsp = plsc.load_gather(dis_v, [idxv])
        d2sp = dsp * dsp
        tdsp = dsp * tk
        for cc in range(_CP // 16):
          w = g0[jj, pl.ds(16 * cc, 16)]
          nn = jj + blk * _DB
          hidc[nn, pl.ds(16 * cc, 16)] = (
              hidc[nn, pl.ds(16 * cc, 16)] + tdsp * w)
          g0[jj, pl.ds(16 * cc, 16)] = d2sp * w
        return c2
      lax.fori_loop(0, _DB, dense_node, 0)
      pltpu.sync_copy(g0, u_s.at[pl.ds(base, _DB)])
      pltpu.sync_copy(g0, acc_s.at[pl.ds(base, _DB)])
    plsc.subcore_barrier()
    return c
  lax.fori_loop(0, _K, hop, 0)

  pltpu.sync_copy(hidc, out_hbm.at[pl.ds(nbase, _NCH)])


def _sc_propagate(h_pad, packed, temp16, interpret=False):
  mesh = plsc.VectorSubcoreMesh(
      core_axis_name="c", subcore_axis_name="s", num_cores=1,
      num_subcores=_NT)
  f = pl.kernel(
      _sc_body,
      out_type=jax.ShapeDtypeStruct((_NP, _CP), jnp.float32),
      mesh=mesh,
      scratch_types=[
          pltpu.VMEM_SHARED((_NP, _CP), jnp.float32),
          pltpu.VMEM_SHARED((_NA, _CP), jnp.float32),
          pltpu.VMEM_SHARED((_NA,), jnp.float32),
          pltpu.VMEM((32, _EB), jnp.int32),
          pltpu.VMEM((32, _EB), jnp.int32),
          pltpu.VMEM((_EB, _CP), jnp.float32),
          pltpu.VMEM((_EB, _CP), jnp.float32),
          pltpu.VMEM((_EB, _CP), jnp.float32),
          pltpu.VMEM((_EB, _CP), jnp.float32),
          pltpu.VMEM((_NCH, _CP), jnp.float32),
          pltpu.VMEM((_NCH,), jnp.float32),
          pltpu.VMEM((_NCH,), jnp.float32),
          pltpu.VMEM((_EB,), jnp.float32),
          pltpu.VMEM((16,), jnp.float32),
          pltpu.SemaphoreType.DMA,
          pltpu.SemaphoreType.DMA,
          pltpu.SemaphoreType.DMA,
          pltpu.SemaphoreType.DMA,
          pltpu.SemaphoreType.DMA,
          pltpu.SemaphoreType.DMA,
          pltpu.SemaphoreType.DMA,
          pltpu.SemaphoreType.DMA,
          pltpu.SemaphoreType.DMA,
          pltpu.SemaphoreType.DMA,
      ],
      compiler_params=pltpu.CompilerParams(needs_layout_passes=False,
                                           use_tc_tiling_on_sc=False),
      interpret=interpret,
  )
  return f(h_pad, packed, temp16)


def _tc_dense(x_pad, W1, b1, W2p, b2p):
  def body(x_ref, w1_ref, b1_ref, w2_ref, b2_ref, o_ref):
    h1 = jnp.dot(x_ref[...], w1_ref[...],
                 preferred_element_type=jnp.float32,
                 precision=lax.Precision.HIGHEST) + b1_ref[...]
    h1 = jnp.maximum(h1, 0.0)
    o_ref[...] = jnp.dot(h1, w2_ref[...],
                         preferred_element_type=jnp.float32,
                         precision=lax.Precision.HIGHEST) + b2_ref[...]

  return pl.pallas_call(
      body,
      grid=(_NP // 256,),
      in_specs=[
          pl.BlockSpec((256, _D), lambda i: (i, 0)),
          pl.BlockSpec((_D, _H), lambda i: (0, 0)),
          pl.BlockSpec((1, _H), lambda i: (0, 0)),
          pl.BlockSpec((_H, _CP), lambda i: (0, 0)),
          pl.BlockSpec((1, _CP), lambda i: (0, 0)),
      ],
      out_specs=pl.BlockSpec((256, _CP), lambda i: (i, 0)),
      out_shape=jax.ShapeDtypeStruct((_NP, _CP), jnp.float32),
  )(x_pad, W1, b1, W2p, b2p)


def _tc_logsoftmax(hid):
  def body(h_ref, o_ref):
    v = h_ref[:, :_C]
    m = jnp.max(v, axis=1, keepdims=True)
    e = jnp.exp(v - m)
    s = jnp.sum(e, axis=1, keepdims=True)
    o_ref[...] = v - m - jnp.log(s)

  return pl.pallas_call(
      body,
      grid=(10,),
      in_specs=[pl.BlockSpec((1000, _CP), lambda i: (i, 0))],
      out_specs=pl.BlockSpec((1000, _C), lambda i: (i, 0)),
      out_shape=jax.ShapeDtypeStruct((_N, _C), jnp.float32),
  )(hid)


@jax.jit
def kernel(x, edge_index, W1, b1, W2, b2, temp):
  x_pad = jnp.zeros((_NP, _D), jnp.float32).at[:_N].set(x)
  W2p = jnp.zeros((_H, _CP), jnp.float32).at[:, :_C].set(W2)
  b2p = jnp.zeros((1, _CP), jnp.float32).at[0, :_C].set(b2)
  row_t = edge_index[0].astype(jnp.int32).reshape(_NT, _ECH)
  col_t = edge_index[1].astype(jnp.int32).reshape(_NT, _ECH)
  pad = _NSB * 16 * _EB - _ECH
  row_sb = jnp.pad(row_t, ((0, 0), (0, pad)),
                   constant_values=0).reshape(_NT, _NSB, 16, _EB)
  col_sb = jnp.pad(col_t, ((0, 0), (0, pad)),
                   constant_values=_NP).reshape(_NT, _NSB, 16, _EB)
  packed = jnp.concatenate([row_sb, col_sb], axis=2)
  temp16 = jnp.zeros((16,), jnp.float32).at[1:_K + 2].set(temp)

  h = _tc_dense(x_pad, W1, b1.reshape(1, _H), W2p, b2p)
  hid = _sc_propagate(h, packed, temp16)
  return _tc_logsoftmax(hid)


# speedup vs baseline: 2.2228x; 1.1187x over previous
import functools

import jax
import jax.numpy as jnp
from jax import lax
from jax.experimental import pallas as pl
from jax.experimental.pallas import tpu as pltpu
from jax.experimental.pallas import tpu_sc as plsc

_N = 10000
_E = 320000
_D = 128
_H = 64
_C = 40
_K = 10

_NP = 10240
_NA = _NP + 16
_CP = 48
_NT = 16
_NCH = _NP // _NT
_ECH = _E // _NT
_EB = 128
_NSB = 10
_NB = _NSB * 16
_DB = 128
_NDB = _NCH // _DB


def _rsqrt(x):
  i = lax.bitcast_convert_type(x, jnp.int32)
  i = jnp.int32(0x5F3759DF) - (i >> 1)
  y = lax.bitcast_convert_type(i, jnp.float32)
  h = x * 0.5
  for _ in range(3):
    y = y * (1.5 - h * y * y)
  return y


def _sc_body(h_hbm, pk_hbm, temp_hbm, out_hbm,
             u_s, acc_s, deg_s,
             ib0, ib1, g0, g1, g2, g3, hidc,
             dis_v, dtmp_v, ones_v, temp_v,
             ise0, ise1, gs0, gs1, gs2, gs3, ss0, ss1, ss2, ss3):
  wid = lax.axis_index("s")
  nbase = wid * _NCH
  gb = (g0, g1, g2, g3)
  gsm = (gs0, gs1, gs2, gs3)
  ssm = (ss0, ss1, ss2, ss3)
  ibs = (ib0, ib1)
  ises = (ise0, ise1)

  pltpu.sync_copy(temp_hbm, temp_v)

  one16 = jnp.full((16,), 1.0, jnp.float32)

  def fill_ones(i, c):
    ones_v[pl.ds(i * 16, 16)] = one16
    return c
  lax.fori_loop(0, _EB // 16, fill_ones, 0)

  def fill_deg(i, c):
    dtmp_v[pl.ds(i * 16, 16)] = one16
    return c
  lax.fori_loop(0, _NCH // 16, fill_deg, 0)
  pltpu.sync_copy(dtmp_v, deg_s.at[pl.ds(nbase, _NCH)])
  plsc.subcore_barrier()

  def deg_sb(sb, c):
    pltpu.sync_copy(pk_hbm.at[wid].at[sb], ib0)
    for j in range(16):
      pltpu.async_copy(ones_v, deg_s.at[ib0.at[16 + j]], ss0, add=True)
    for j in range(16):
      pltpu.make_async_copy(ones_v, deg_s.at[ib0.at[16 + j]], ss0).wait()
    return c
  lax.fori_loop(0, _NSB, deg_sb, 0)
  plsc.subcore_barrier()

  pltpu.sync_copy(deg_s.at[pl.ds(nbase, _NCH)], dtmp_v)

  def mk_dis(i, c):
    d = dtmp_v[pl.ds(i * 16, 16)]
    dis_v[pl.ds(i * 16, 16)] = _rsqrt(d)
    return c
  lax.fori_loop(0, _NCH // 16, mk_dis, 0)

  t0 = plsc.load_gather(temp_v, [jnp.full((16,), 1, jnp.int32)])
  for blk in range(_NDB):
    base = nbase + blk * _DB
    pltpu.sync_copy(h_hbm.at[pl.ds(base, _DB)], g0)

    @plsc.parallel_loop(0, _DB, unroll=4)
    def init_node(jj, blk=blk):
      idxv = jnp.full((16,), jj + blk * _DB, jnp.int32)
      dsp = plsc.load_gather(dis_v, [idxv])
      for cc in range(_CP // 16):
        hcol = g0[jj, pl.ds(16 * cc, 16)]
        hidc[jj + blk * _DB, pl.ds(16 * cc, 16)] = t0 * hcol
        g0[jj, pl.ds(16 * cc, 16)] = dsp * hcol
    pltpu.sync_copy(g0, u_s.at[pl.ds(base, _DB)])
    pltpu.sync_copy(g0, acc_s.at[pl.ds(base, _DB)])
  plsc.subcore_barrier()

  def hop(k, c):
    pltpu.sync_copy(pk_hbm.at[wid].at[0], ib0)
    pltpu.async_copy(u_s.at[ib0.at[0]], g0, gs0)
    pltpu.async_copy(u_s.at[ib0.at[1]], g1, gs1)

    def sb_pair(p, c2):
      for q in range(2):
        sb = 2 * p + q
        ibA = ibs[q]
        ibB = ibs[1 - q]
        iseB = ises[1 - q]
        for j in range(16):
          m = j % 4
          pltpu.make_async_copy(u_s.at[ibA.at[j]], gb[m], gsm[m]).wait()
          pltpu.async_copy(gb[m], acc_s.at[ibA.at[16 + j]], ssm[m],
                           add=True)
          if j == 2:
            @pl.when(sb < _NSB - 1)
            def _():
              pltpu.async_copy(pk_hbm.at[wid].at[sb + 1], ibB, iseB)
          nj = j + 2
          m2 = nj % 4
          gg = sb * 16 + j

          @pl.when(gg + 2 < _NB)
          def _(nj=nj, m2=m2, gg=gg, ibA=ibA, ibB=ibB, iseB=iseB):
            @pl.when(gg >= 2)
            def __():
              pltpu.make_async_copy(gb[m2], acc_s.at[ibA.at[16]],
                                    ssm[m2]).wait()
            if nj < 16:
              pltpu.async_copy(u_s.at[ibA.at[nj]], gb[m2], gsm[m2])
            else:
              if nj == 16:
                pltpu.make_async_copy(pk_hbm.at[wid].at[0], ibB,
                                      iseB).wait()
              pltpu.async_copy(u_s.at[ibB.at[nj - 16]], gb[m2], gsm[m2])
      return c2
    lax.fori_loop(0, _NSB // 2, sb_pair, 0)
    pltpu.make_async_copy(g0, acc_s.at[ib0.at[16]], ss0).wait()
    pltpu.make_async_copy(g1, acc_s.at[ib0.at[16]], ss1).wait()
    pltpu.make_async_copy(g2, acc_s.at[ib0.at[16]], ss2).wait()
    pltpu.make_async_copy(g3, acc_s.at[ib0.at[16]], ss3).wait()
    plsc.subcore_barrier()

    tk = plsc.load_gather(temp_v, [jnp.full((16,), 2, jnp.int32) + k])
    pltpu.async_copy(acc_s.at[pl.ds(nbase, _DB)], g0, gs0)
    for blk in range(_NDB):
      base = nbase + blk * _DB
      stg = gb[blk % 2]
      ssem = gsm[blk % 2]
      pltpu.make_async_copy(acc_s.at[pl.ds(base, _DB)], stg, ssem).wait()
      if blk + 1 < _NDB:
        pltpu.async_copy(acc_s.at[pl.ds(base + _DB, _DB)], gb[(blk + 1) % 2],
                         gsm[(blk + 1) % 2])

      @plsc.parallel_loop(0, _DB, unroll=4)
      def dense_node(jj, blk=blk, tk=tk, stg=stg):
        idxv = jnp.full((16,), jj + blk * _DB, jnp.int32)
        dsp = plsc.load_gather(dis_v, [idxv])
        d2sp = dsp * dsp
        tdsp = dsp * tk
        for cc in range(_CP // 16):
          w = stg[jj, pl.ds(16 * cc, 16)]
          nn = jj + blk * _DB
          hidc[nn, pl.ds(16 * cc, 16)] = (
              hidc[nn, pl.ds(16 * cc, 16)] + tdsp * w)
          stg[jj, pl.ds(16 * cc, 16)] = d2sp * w

      pltpu.async_copy(stg, u_s.at[pl.ds(base, _DB)], gsm[2 + blk % 2])
      pltpu.sync_copy(stg, acc_s.at[pl.ds(base, _DB)])
      pltpu.make_async_copy(stg, u_s.at[pl.ds(base, _DB)],
                            gsm[2 + blk % 2]).wait()
    plsc.subcore_barrier()
    return c
  lax.fori_loop(0, _K, hop, 0)

  pltpu.sync_copy(hidc, out_hbm.at[pl.ds(nbase, _NCH)])


def _sc_propagate(h_pad, packed, temp16, interpret=False):
  mesh = plsc.VectorSubcoreMesh(
      core_axis_name="c", subcore_axis_name="s", num_cores=1,
      num_subcores=_NT)
  f = pl.kernel(
      _sc_body,
      out_type=jax.ShapeDtypeStruct((_NP, _CP), jnp.float32),
      mesh=mesh,
      scratch_types=[
          pltpu.VMEM_SHARED((_NP, _CP), jnp.float32),
          pltpu.VMEM_SHARED((_NA, _CP), jnp.float32),
          pltpu.VMEM_SHARED((_NA,), jnp.float32),
          pltpu.VMEM((32, _EB), jnp.int32),
          pltpu.VMEM((32, _EB), jnp.int32),
          pltpu.VMEM((_EB, _CP), jnp.float32),
          pltpu.VMEM((_EB, _CP), jnp.float32),
          pltpu.VMEM((_EB, _CP), jnp.float32),
          pltpu.VMEM((_EB, _CP), jnp.float32),
          pltpu.VMEM((_NCH, _CP), jnp.float32),
          pltpu.VMEM((_NCH,), jnp.float32),
          pltpu.VMEM((_NCH,), jnp.float32),
          pltpu.VMEM((_EB,), jnp.float32),
          pltpu.VMEM((16,), jnp.float32),
          pltpu.SemaphoreType.DMA,
          pltpu.SemaphoreType.DMA,
          pltpu.SemaphoreType.DMA,
          pltpu.SemaphoreType.DMA,
          pltpu.SemaphoreType.DMA,
          pltpu.SemaphoreType.DMA,
          pltpu.SemaphoreType.DMA,
          pltpu.SemaphoreType.DMA,
          pltpu.SemaphoreType.DMA,
          pltpu.SemaphoreType.DMA,
      ],
      compiler_params=pltpu.CompilerParams(needs_layout_passes=False,
                                           use_tc_tiling_on_sc=False),
      interpret=interpret,
  )
  return f(h_pad, packed, temp16)


def _tc_dense(x_pad, W1, b1, W2p, b2p):
  def body(x_ref, w1_ref, b1_ref, w2_ref, b2_ref, o_ref):
    h1 = jnp.dot(x_ref[...], w1_ref[...],
                 preferred_element_type=jnp.float32,
                 precision=lax.Precision.HIGHEST) + b1_ref[...]
    h1 = jnp.maximum(h1, 0.0)
    o_ref[...] = jnp.dot(h1, w2_ref[...],
                         preferred_element_type=jnp.float32,
                         precision=lax.Precision.HIGHEST) + b2_ref[...]

  return pl.pallas_call(
      body,
      grid=(_NP // 256,),
      in_specs=[
          pl.BlockSpec((256, _D), lambda i: (i, 0)),
          pl.BlockSpec((_D, _H), lambda i: (0, 0)),
          pl.BlockSpec((1, _H), lambda i: (0, 0)),
          pl.BlockSpec((_H, _CP), lambda i: (0, 0)),
          pl.BlockSpec((1, _CP), lambda i: (0, 0)),
      ],
      out_specs=pl.BlockSpec((256, _CP), lambda i: (i, 0)),
      out_shape=jax.ShapeDtypeStruct((_NP, _CP), jnp.float32),
  )(x_pad, W1, b1, W2p, b2p)


def _tc_logsoftmax(hid):
  def body(h_ref, o_ref):
    v = h_ref[:, :_C]
    m = jnp.max(v, axis=1, keepdims=True)
    e = jnp.exp(v - m)
    s = jnp.sum(e, axis=1, keepdims=True)
    o_ref[...] = v - m - jnp.log(s)

  return pl.pallas_call(
      body,
      grid=(10,),
      in_specs=[pl.BlockSpec((1000, _CP), lambda i: (i, 0))],
      out_specs=pl.BlockSpec((1000, _C), lambda i: (i, 0)),
      out_shape=jax.ShapeDtypeStruct((_N, _C), jnp.float32),
  )(hid)


@jax.jit
def kernel(x, edge_index, W1, b1, W2, b2, temp):
  x_pad = jnp.zeros((_NP, _D), jnp.float32).at[:_N].set(x)
  W2p = jnp.zeros((_H, _CP), jnp.float32).at[:, :_C].set(W2)
  b2p = jnp.zeros((1, _CP), jnp.float32).at[0, :_C].set(b2)
  row_t = edge_index[0].astype(jnp.int32).reshape(_NT, _ECH)
  col_t = edge_index[1].astype(jnp.int32).reshape(_NT, _ECH)
  pad = _NSB * 16 * _EB - _ECH
  row_sb = jnp.pad(row_t, ((0, 0), (0, pad)),
                   constant_values=0).reshape(_NT, _NSB, 16, _EB)
  col_sb = jnp.pad(col_t, ((0, 0), (0, pad)),
                   constant_values=_NP).reshape(_NT, _NSB, 16, _EB)
  packed = jnp.concatenate([row_sb, col_sb], axis=2)
  temp16 = jnp.zeros((16,), jnp.float32).at[1:_K + 2].set(temp)

  h = _tc_dense(x_pad, W1, b1.reshape(1, _H), W2p, b2p)
  hid = _sc_propagate(h, packed, temp16)
  return _tc_logsoftmax(hid)


# speedup vs baseline: 2.2509x; 1.0127x over previous
import functools

import jax
import jax.numpy as jnp
from jax import lax
from jax.experimental import pallas as pl
from jax.experimental.pallas import tpu as pltpu
from jax.experimental.pallas import tpu_sc as plsc

_N = 10000
_E = 320000
_D = 128
_H = 64
_C = 40
_K = 10

_NP = 10240
_NA = _NP + 16
_CP = 48
_NT = 16
_NCH = _NP // _NT
_ECH = _E // _NT
_EB = 128
_NSB = 10
_NB = _NSB * 16
_DB = 128
_NDB = _NCH // _DB


def _rsqrt(x):
  i = lax.bitcast_convert_type(x, jnp.int32)
  i = jnp.int32(0x5F3759DF) - (i >> 1)
  y = lax.bitcast_convert_type(i, jnp.float32)
  h = x * 0.5
  for _ in range(3):
    y = y * (1.5 - h * y * y)
  return y


def _sc_body(h_hbm, pk_hbm, temp_hbm, out_hbm,
             u_s, acc_s, deg_s,
             ib0, ib1, g0, g1, g2, g3, hidc,
             dis_v, dtmp_v, ones_v, temp_v,
             ise0, ise1, gs0, gs1, gs2, gs3, ss0, ss1, ss2, ss3):
  wid = lax.axis_index("s")
  nbase = wid * _NCH
  gb = (g0, g1, g2, g3)
  gsm = (gs0, gs1, gs2, gs3)
  ssm = (ss0, ss1, ss2, ss3)
  ibs = (ib0, ib1)
  ises = (ise0, ise1)

  pltpu.sync_copy(temp_hbm, temp_v)

  one16 = jnp.full((16,), 1.0, jnp.float32)

  def fill_ones(i, c):
    ones_v[pl.ds(i * 16, 16)] = one16
    return c
  lax.fori_loop(0, _EB // 16, fill_ones, 0)

  def fill_deg(i, c):
    dtmp_v[pl.ds(i * 16, 16)] = one16
    return c
  lax.fori_loop(0, _NCH // 16, fill_deg, 0)
  pltpu.sync_copy(dtmp_v, deg_s.at[pl.ds(nbase, _NCH)])
  plsc.subcore_barrier()

  pltpu.sync_copy(pk_hbm.at[wid].at[0], ib0)
  pltpu.async_copy(pk_hbm.at[wid].at[1], ib1, ise1)

  def deg_pair(p, c):
    for q in range(2):
      sb = 2 * p + q
      ibA = ibs[q]
      iseA = ises[q]

      @pl.when(sb > 0)
      def _(ibA=ibA, iseA=iseA):
        pltpu.make_async_copy(pk_hbm.at[wid].at[0], ibA, iseA).wait()
      for j in range(16):
        pltpu.async_copy(ones_v, deg_s.at[ibA.at[16 + j]], ss0, add=True)
      for j in range(16):
        pltpu.make_async_copy(ones_v, deg_s.at[ibA.at[16 + j]], ss0).wait()

      @pl.when(sb + 2 < _NSB)
      def _(sb=sb, ibA=ibA, iseA=iseA):
        pltpu.async_copy(pk_hbm.at[wid].at[sb + 2], ibA, iseA)
    return c
  lax.fori_loop(0, _NSB // 2, deg_pair, 0)
  plsc.subcore_barrier()

  pltpu.sync_copy(deg_s.at[pl.ds(nbase, _NCH)], dtmp_v)

  def mk_dis(i, c):
    d = dtmp_v[pl.ds(i * 16, 16)]
    dis_v[pl.ds(i * 16, 16)] = _rsqrt(d)
    return c
  lax.fori_loop(0, _NCH // 16, mk_dis, 0)

  t0 = plsc.load_gather(temp_v, [jnp.full((16,), 1, jnp.int32)])
  for blk in range(_NDB):
    base = nbase + blk * _DB
    pltpu.sync_copy(h_hbm.at[pl.ds(base, _DB)], g0)

    @plsc.parallel_loop(0, _DB, unroll=4)
    def init_node(jj, blk=blk):
      idxv = jnp.full((16,), jj + blk * _DB, jnp.int32)
      dsp = plsc.load_gather(dis_v, [idxv])
      for cc in range(_CP // 16):
        hcol = g0[jj, pl.ds(16 * cc, 16)]
        hidc[jj + blk * _DB, pl.ds(16 * cc, 16)] = t0 * hcol
        g0[jj, pl.ds(16 * cc, 16)] = dsp * hcol
    pltpu.sync_copy(g0, u_s.at[pl.ds(base, _DB)])
    pltpu.sync_copy(g0, acc_s.at[pl.ds(base, _DB)])
  pltpu.async_copy(pk_hbm.at[wid].at[0], ib0, ise0)
  plsc.subcore_barrier()

  def hop(k, c):
    pltpu.make_async_copy(pk_hbm.at[wid].at[0], ib0, ise0).wait()
    pltpu.async_copy(u_s.at[ib0.at[0]], g0, gs0)
    pltpu.async_copy(u_s.at[ib0.at[1]], g1, gs1)

    def sb_pair(p, c2):
      for q in range(2):
        sb = 2 * p + q
        ibA = ibs[q]
        ibB = ibs[1 - q]
        iseB = ises[1 - q]
        for j in range(16):
          m = j % 4
          pltpu.make_async_copy(u_s.at[ibA.at[j]], gb[m], gsm[m]).wait()
          pltpu.async_copy(gb[m], acc_s.at[ibA.at[16 + j]], ssm[m],
                           add=True)
          if j == 2:
            @pl.when(sb < _NSB - 1)
            def _():
              pltpu.async_copy(pk_hbm.at[wid].at[sb + 1], ibB, iseB)
          nj = j + 2
          m2 = nj % 4
          gg = sb * 16 + j

          @pl.when(gg + 2 < _NB)
          def _(nj=nj, m2=m2, gg=gg, ibA=ibA, ibB=ibB, iseB=iseB):
            @pl.when(gg >= 2)
            def __():
              pltpu.make_async_copy(gb[m2], acc_s.at[ibA.at[16]],
                                    ssm[m2]).wait()
            if nj < 16:
              pltpu.async_copy(u_s.at[ibA.at[nj]], gb[m2], gsm[m2])
            else:
              if nj == 16:
                pltpu.make_async_copy(pk_hbm.at[wid].at[0], ibB,
                                      iseB).wait()
              pltpu.async_copy(u_s.at[ibB.at[nj - 16]], gb[m2], gsm[m2])
      return c2
    lax.fori_loop(0, _NSB // 2, sb_pair, 0)
    pltpu.make_async_copy(g0, acc_s.at[ib0.at[16]], ss0).wait()
    pltpu.make_async_copy(g1, acc_s.at[ib0.at[16]], ss1).wait()
    pltpu.make_async_copy(g2, acc_s.at[ib0.at[16]], ss2).wait()
    pltpu.make_async_copy(g3, acc_s.at[ib0.at[16]], ss3).wait()
    plsc.subcore_barrier()

    @pl.when(k < _K - 1)
    def _():
      pltpu.async_copy(pk_hbm.at[wid].at[0], ib0, ise0)
    tk = plsc.load_gather(temp_v, [jnp.full((16,), 2, jnp.int32) + k])
    pltpu.async_copy(acc_s.at[pl.ds(nbase, _DB)], g0, gs0)
    for blk in range(_NDB):
      base = nbase + blk * _DB
      stg = gb[blk % 2]
      ssem = gsm[blk % 2]
      pltpu.make_async_copy(acc_s.at[pl.ds(base, _DB)], stg, ssem).wait()
      if blk + 1 < _NDB:
        pltpu.async_copy(acc_s.at[pl.ds(base + _DB, _DB)], gb[(blk + 1) % 2],
                         gsm[(blk + 1) % 2])

      @plsc.parallel_loop(0, _DB, unroll=4)
      def dense_node(jj, blk=blk, tk=tk, stg=stg):
        idxv = jnp.full((16,), jj + blk * _DB, jnp.int32)
        dsp = plsc.load_gather(dis_v, [idxv])
        d2sp = dsp * dsp
        tdsp = dsp * tk
        for cc in range(_CP // 16):
          w = stg[jj, pl.ds(16 * cc, 16)]
          nn = jj + blk * _DB
          hidc[nn, pl.ds(16 * cc, 16)] = (
              hidc[nn, pl.ds(16 * cc, 16)] + tdsp * w)
          stg[jj, pl.ds(16 * cc, 16)] = d2sp * w

      pltpu.async_copy(stg, u_s.at[pl.ds(base, _DB)], gsm[2 + blk % 2])
      pltpu.sync_copy(stg, acc_s.at[pl.ds(base, _DB)])
      pltpu.make_async_copy(stg, u_s.at[pl.ds(base, _DB)],
                            gsm[2 + blk % 2]).wait()
    plsc.subcore_barrier()
    return c
  lax.fori_loop(0, _K, hop, 0)

  pltpu.sync_copy(hidc, out_hbm.at[pl.ds(nbase, _NCH)])


def _sc_propagate(h_pad, packed, temp16, interpret=False):
  mesh = plsc.VectorSubcoreMesh(
      core_axis_name="c", subcore_axis_name="s", num_cores=1,
      num_subcores=_NT)
  f = pl.kernel(
      _sc_body,
      out_type=jax.ShapeDtypeStruct((_NP, _CP), jnp.float32),
      mesh=mesh,
      scratch_types=[
          pltpu.VMEM_SHARED((_NP, _CP), jnp.float32),
          pltpu.VMEM_SHARED((_NA, _CP), jnp.float32),
          pltpu.VMEM_SHARED((_NA,), jnp.float32),
          pltpu.VMEM((32, _EB), jnp.int32),
          pltpu.VMEM((32, _EB), jnp.int32),
          pltpu.VMEM((_EB, _CP), jnp.float32),
          pltpu.VMEM((_EB, _CP), jnp.float32),
          pltpu.VMEM((_EB, _CP), jnp.float32),
          pltpu.VMEM((_EB, _CP), jnp.float32),
          pltpu.VMEM((_NCH, _CP), jnp.float32),
          pltpu.VMEM((_NCH,), jnp.float32),
          pltpu.VMEM((_NCH,), jnp.float32),
          pltpu.VMEM((_EB,), jnp.float32),
          pltpu.VMEM((16,), jnp.float32),
          pltpu.SemaphoreType.DMA,
          pltpu.SemaphoreType.DMA,
          pltpu.SemaphoreType.DMA,
          pltpu.SemaphoreType.DMA,
          pltpu.SemaphoreType.DMA,
          pltpu.SemaphoreType.DMA,
          pltpu.SemaphoreType.DMA,
          pltpu.SemaphoreType.DMA,
          pltpu.SemaphoreType.DMA,
          pltpu.SemaphoreType.DMA,
      ],
      compiler_params=pltpu.CompilerParams(needs_layout_passes=False,
                                           use_tc_tiling_on_sc=False),
      interpret=interpret,
  )
  return f(h_pad, packed, temp16)


def _tc_dense(x_pad, W1, b1, W2p, b2p):
  def body(x_ref, w1_ref, b1_ref, w2_ref, b2_ref, o_ref):
    h1 = jnp.dot(x_ref[...], w1_ref[...],
                 preferred_element_type=jnp.float32,
                 precision=lax.Precision.HIGHEST) + b1_ref[...]
    h1 = jnp.maximum(h1, 0.0)
    o_ref[...] = jnp.dot(h1, w2_ref[...],
                         preferred_element_type=jnp.float32,
                         precision=lax.Precision.HIGHEST) + b2_ref[...]

  return pl.pallas_call(
      body,
      grid=(_NP // 256,),
      in_specs=[
          pl.BlockSpec((256, _D), lambda i: (i, 0)),
          pl.BlockSpec((_D, _H), lambda i: (0, 0)),
          pl.BlockSpec((1, _H), lambda i: (0, 0)),
          pl.BlockSpec((_H, _CP), lambda i: (0, 0)),
          pl.BlockSpec((1, _CP), lambda i: (0, 0)),
      ],
      out_specs=pl.BlockSpec((256, _CP), lambda i: (i, 0)),
      out_shape=jax.ShapeDtypeStruct((_NP, _CP), jnp.float32),
  )(x_pad, W1, b1, W2p, b2p)


def _tc_logsoftmax(hid):
  def body(h_ref, o_ref):
    v = h_ref[:, :_C]
    m = jnp.max(v, axis=1, keepdims=True)
    e = jnp.exp(v - m)
    s = jnp.sum(e, axis=1, keepdims=True)
    o_ref[...] = v - m - jnp.log(s)

  return pl.pallas_call(
      body,
      grid=(10,),
      in_specs=[pl.BlockSpec((1000, _CP), lambda i: (i, 0))],
      out_specs=pl.BlockSpec((1000, _C), lambda i: (i, 0)),
      out_shape=jax.ShapeDtypeStruct((_N, _C), jnp.float32),
  )(hid)


@jax.jit
def kernel(x, edge_index, W1, b1, W2, b2, temp):
  x_pad = jnp.zeros((_NP, _D), jnp.float32).at[:_N].set(x)
  W2p = jnp.zeros((_H, _CP), jnp.float32).at[:, :_C].set(W2)
  b2p = jnp.zeros((1, _CP), jnp.float32).at[0, :_C].set(b2)
  row_t = edge_index[0].astype(jnp.int32).reshape(_NT, _ECH)
  col_t = edge_index[1].astype(jnp.int32).reshape(_NT, _ECH)
  pad = _NSB * 16 * _EB - _ECH
  row_sb = jnp.pad(row_t, ((0, 0), (0, pad)),
                   constant_values=0).reshape(_NT, _NSB, 16, _EB)
  col_sb = jnp.pad(col_t, ((0, 0), (0, pad)),
                   constant_values=_NP).reshape(_NT, _NSB, 16, _EB)
  packed = jnp.concatenate([row_sb, col_sb], axis=2)
  temp16 = jnp.zeros((16,), jnp.float32).at[1:_K + 2].set(temp)

  h = _tc_dense(x_pad, W1, b1.reshape(1, _H), W2p, b2p)
  hid = _sc_propagate(h, packed, temp16)
  return _tc_logsoftmax(hid)
